# Initial kernel scaffold; baseline (speedup 1.0000x reference)
import jax
import jax.numpy as jnp
from jax.experimental import pallas as pl


def kernel(x, edge_index, W1_l, b1, W1_r, W2_l, b2, W2_r):
    raise NotImplementedError("write your pallas kernel here")


# speedup vs baseline: 3.5054x; 3.5054x over previous
import functools

import jax
import jax.numpy as jnp
from jax import lax
from jax.experimental import pallas as pl
from jax.experimental.pallas import tpu as pltpu
from jax.experimental.pallas import tpu_sc as plsc

N_NODES = 10000
N_SUBCORES = 16
CHUNK = 128
ACC_ROWS = 10112
ZROWS = ACC_ROWS // N_SUBCORES


def _make_segsum(width, n_chunks):
    mesh = plsc.VectorSubcoreMesh(core_axis_name="c", subcore_axis_name="s")

    @functools.partial(
        pl.kernel,
        out_type=[
            jax.ShapeDtypeStruct((ACC_ROWS, width), jnp.float32),
            jax.ShapeDtypeStruct((ACC_ROWS, width), jnp.float32),
        ],
        mesh=mesh,
        compiler_params=pltpu.CompilerParams(use_tc_tiling_on_sc=False),
        scratch_types=[
            pltpu.VMEM((n_chunks, CHUNK), jnp.int32),
            pltpu.VMEM((n_chunks, CHUNK), jnp.int32),
            pltpu.VMEM((CHUNK, width), jnp.float32),
            pltpu.VMEM_SHARED((ACC_ROWS, width), jnp.float32),
            pltpu.SemaphoreType.DMA,
        ],
    )
    def segsum(tab0, tab1, srcs, dsts, zeros, out0, out1,
               src_v, dst_v, rows_v, acc, sem):
        c = lax.axis_index("c")
        s = lax.axis_index("s")
        pltpu.sync_copy(zeros, acc.at[pl.ds(s * ZROWS, ZROWS)])
        pltpu.sync_copy(srcs.at[s], src_v)
        pltpu.sync_copy(dsts.at[s], dst_v)
        plsc.subcore_barrier()

        def run(tab, out):
            def body(j, carry):
                pltpu.async_copy(tab.at[src_v.at[j]], rows_v, sem).wait()
                pltpu.sync_copy(rows_v, acc.at[dst_v.at[j]], add=True)
                return carry
            lax.fori_loop(0, n_chunks, body, 0)
            plsc.subcore_barrier()
            pltpu.sync_copy(acc.at[pl.ds(s * ZROWS, ZROWS)],
                            out.at[pl.ds(s * ZROWS, ZROWS)])

        @pl.when(c == 0)
        def _():
            run(tab0, out0)

        @pl.when(c == 1)
        def _():
            run(tab1, out1)

    return segsum


def _dense_body(agg_ref, xr_ref, d_ref, wl_ref, wr_ref, b_ref, o_ref, *, relu):
    inv = 1.0 / jnp.maximum(d_ref[...], 1.0)
    acc = jnp.dot(agg_ref[...] * inv, wl_ref[...],
                  preferred_element_type=jnp.float32)
    acc = acc + jnp.dot(xr_ref[...], wr_ref[...],
                        preferred_element_type=jnp.float32)
    acc = acc + b_ref[...]
    o_ref[...] = jnp.maximum(acc, 0.0) if relu else acc


def _dense_layer(agg, xr, dcol, wl, wr, bias, relu, mb=1000):
    m, k = agg.shape
    k2 = xr.shape[1]
    n = wl.shape[1]
    return pl.pallas_call(
        functools.partial(_dense_body, relu=relu),
        grid=(m // mb,),
        in_specs=[
            pl.BlockSpec((mb, k), lambda i: (i, 0)),
            pl.BlockSpec((mb, k2), lambda i: (i, 0)),
            pl.BlockSpec((mb, 1), lambda i: (i, 0)),
            pl.BlockSpec((k, n), lambda i: (0, 0)),
            pl.BlockSpec((k2, n), lambda i: (0, 0)),
            pl.BlockSpec((1, n), lambda i: (0, 0)),
        ],
        out_specs=pl.BlockSpec((mb, n), lambda i: (i, 0)),
        out_shape=jax.ShapeDtypeStruct((m, n), jnp.float32),
    )(agg, xr, dcol, wl, wr, bias)


def kernel(x, edge_index, W1_l, b1, W1_r, W2_l, b2, W2_r):
    src = edge_index[0].astype(jnp.int32)
    dst = edge_index[1].astype(jnp.int32)
    n_edges = src.shape[0]

    per_tile = -(-n_edges // N_SUBCORES)
    n_chunks = -(-per_tile // CHUNK)
    e_pad = N_SUBCORES * n_chunks * CHUNK - n_edges
    srcs = jnp.concatenate([src, jnp.zeros((e_pad,), jnp.int32)])
    dsts = jnp.concatenate([dst, jnp.full((e_pad,), N_NODES, jnp.int32)])
    srcs = srcs.reshape(N_SUBCORES, n_chunks, CHUNK)
    dsts = dsts.reshape(N_SUBCORES, n_chunks, CHUNK)

    tab0 = x[:, :144]
    tab1 = jnp.concatenate(
        [x[:, 144:], jnp.ones((N_NODES, 32), jnp.float32)], axis=1)
    z144 = jnp.zeros((ZROWS, 144), jnp.float32)
    agg_a, agg_b = _make_segsum(144, n_chunks)(tab0, tab1, srcs, dsts, z144)
    agg1 = jnp.concatenate([agg_a[:N_NODES], agg_b[:N_NODES, :112]], axis=1)
    dcol = agg_b[:N_NODES, 112:113]

    h = _dense_layer(agg1, x, dcol, W1_l, W1_r, b1.reshape(1, -1), relu=True)

    z128 = jnp.zeros((ZROWS, 128), jnp.float32)
    seg128 = _make_segsum(128, n_chunks)
    a20, a21 = seg128(h[:, 0:128], h[:, 128:256], srcs, dsts, z128)
    a22, a23 = seg128(h[:, 256:384], h[:, 384:512], srcs, dsts, z128)
    agg2 = jnp.concatenate(
        [a20[:N_NODES], a21[:N_NODES], a22[:N_NODES], a23[:N_NODES]], axis=1)

    out = _dense_layer(agg2, h, dcol, W2_l, W2_r, b2.reshape(1, -1),
                       relu=False)
    return out
